# Initial kernel scaffold; baseline (speedup 1.0000x reference)
import jax
import jax.numpy as jnp
from jax.experimental import pallas as pl


def kernel(features, adj, idx, W1, b1, W2, b2, dW1, db1, dW2, db2):
    raise NotImplementedError("write your pallas kernel here")


# speedup vs baseline: 4.4679x; 4.4679x over previous
import functools

import jax
import jax.numpy as jnp
from jax import lax
from jax.experimental import pallas as pl
from jax.experimental.pallas import tpu as pltpu
from jax.experimental.pallas import tpu_sc as plsc

N = 10000
NPAD = 10240
E = 320000
NW = 32
ECH = 80
EPT = ECH * 128
EPAD = NW * EPT
P = 100000
PCH = 25
PPT = PCH * 128
PPAD = NW * PPT
DEGW = 16
STRIPE = NPAD // 16

_mesh = lambda: plsc.VectorSubcoreMesh(core_axis_name="c", subcore_axis_name="s")


def _make_scatter_kernel(D):
    @functools.partial(
        pl.kernel,
        mesh=_mesh(),
        out_type=jax.ShapeDtypeStruct((2, NPAD, D), jnp.float32),
        scratch_types=[
            pltpu.VMEM((ECH, 128), jnp.int32),
            pltpu.VMEM((ECH, 128), jnp.int32),
            pltpu.VMEM((128, D), jnp.float32),
            pltpu.VMEM_SHARED((NPAD, D), jnp.float32),
            pltpu.SemaphoreType.DMA,
        ],
    )
    def k(table_hbm, src_hbm, dst_hbm, zeros_hbm, out_hbm,
          src_v, dst_v, rows_v, acc_s, sem):
        cid = lax.axis_index("c")
        sid = lax.axis_index("s")
        wid = cid * 16 + sid
        pltpu.sync_copy(src_hbm.at[wid], src_v)
        pltpu.sync_copy(dst_hbm.at[wid], dst_v)
        pltpu.sync_copy(zeros_hbm, rows_v)
        for kk in range(STRIPE // 128):
            pltpu.sync_copy(rows_v, acc_s.at[pl.ds(sid * STRIPE + kk * 128, 128)])
        plsc.subcore_barrier()

        def body(j, carry):
            pltpu.async_copy(table_hbm.at[src_v.at[j]], rows_v, sem).wait()
            pltpu.sync_copy(rows_v, acc_s.at[dst_v.at[j]], add=True)
            return carry

        lax.fori_loop(0, ECH, body, 0)
        plsc.subcore_barrier()
        for kk in range(STRIPE // 128):
            pltpu.sync_copy(acc_s.at[pl.ds(sid * STRIPE + kk * 128, 128)], rows_v)
            pltpu.sync_copy(rows_v, out_hbm.at[cid, pl.ds(sid * STRIPE + kk * 128, 128)])

    return k


_scatter128 = _make_scatter_kernel(128)


@functools.partial(
    pl.kernel,
    mesh=_mesh(),
    out_type=jax.ShapeDtypeStruct((2, PPAD, 128), jnp.float32),
    scratch_types=[
        pltpu.VMEM((PCH, 128), jnp.int32),
        pltpu.VMEM((PCH, 128), jnp.int32),
        pltpu.VMEM((128, 128), jnp.float32),
        pltpu.SemaphoreType.DMA,
    ],
)
def _pair_kernel(table_hbm, i0_hbm, i1_hbm, out_hbm, i0_v, i1_v, rows_v, sem):
    cid = lax.axis_index("c")
    sid = lax.axis_index("s")
    wid = cid * 16 + sid
    base = wid * PPT
    pltpu.sync_copy(i0_hbm.at[wid], i0_v)
    pltpu.sync_copy(i1_hbm.at[wid], i1_v)

    def body(j, carry):
        pltpu.async_copy(table_hbm.at[i0_v.at[j]], rows_v, sem).wait()
        pltpu.sync_copy(rows_v, out_hbm.at[0, pl.ds(base + j * 128, 128)])
        pltpu.async_copy(table_hbm.at[i1_v.at[j]], rows_v, sem).wait()
        pltpu.sync_copy(rows_v, out_hbm.at[1, pl.ds(base + j * 128, 128)])
        return carry

    lax.fori_loop(0, PCH, body, 0)


def _mm1_body(dp_ref, f_ref, w_ref, x1s_ref, dinv_ref):
    deg = dp_ref[:, 0:1] + dp_ref[:, 1:2] + 1.0
    dinv = lax.rsqrt(jnp.maximum(deg, 1e-12))
    x1 = jnp.dot(f_ref[...], w_ref[...], preferred_element_type=jnp.float32)
    x1s_ref[...] = x1 * dinv
    dinv_ref[...] = dinv


def _l2_body(p_ref, x1s_ref, dinv_ref, b1_ref, w2_ref, x2s_ref):
    s = p_ref[0] + p_ref[1] + x1s_ref[...]
    h1 = jnp.maximum(dinv_ref[...] * s + b1_ref[...], 0.0)
    y = jnp.dot(h1, w2_ref[...], preferred_element_type=jnp.float32) * dinv_ref[...]
    x2s_ref[...] = jnp.concatenate([y, jnp.zeros_like(y)], axis=1)


def _h2_body(q_ref, x2s_ref, dinv_ref, b2_ref, h2_ref):
    s = q_ref[0] + q_ref[1] + x2s_ref[...]
    h2_ref[...] = jnp.maximum(dinv_ref[...] * s + b2_ref[...], 0.0)


def _dec_body(xj_ref, xi_ref, dw1_ref, db1_ref, dw2_ref, db2_ref, o_ref):
    h = jnp.abs(xj_ref[...] - xi_ref[...])
    g = jax.nn.sigmoid(
        jnp.dot(h, dw1_ref[...], preferred_element_type=jnp.float32) + db1_ref[...])
    o_ref[...] = jnp.dot(g, dw2_ref[...], preferred_element_type=jnp.float32) + db2_ref[...]


def _pad_reshape_idx(a, total, per_tile_chunks, fill):
    pad = total - a.shape[0]
    a32 = a.astype(jnp.int32)
    return jnp.concatenate(
        [a32, jnp.full((pad,), fill, jnp.int32)]).reshape(NW, per_tile_chunks, 128)


def kernel(features, adj, idx, W1, b1, W2, b2, dW1, db1, dW2, db2):
    src_r = _pad_reshape_idx(adj[0], EPAD, ECH, 0)
    dst_r = _pad_reshape_idx(adj[1], EPAD, ECH, N)
    i0_r = _pad_reshape_idx(idx[0], PPAD, PCH, 0)
    i1_r = _pad_reshape_idx(idx[1], PPAD, PCH, 0)
    feat_pad = jnp.pad(features, ((0, NPAD - N), (0, 0)))

    zeros128 = jnp.zeros((128, 128), jnp.float32)
    ones_table = jnp.ones((NPAD, 128), jnp.float32)

    deg_parts = _scatter128(ones_table, src_r, dst_r, zeros128)
    dp_t = jnp.transpose(deg_parts[:, :, 0])

    x1s, dinv = pl.pallas_call(
        _mm1_body,
        out_shape=[jax.ShapeDtypeStruct((NPAD, 128), jnp.float32),
                   jax.ShapeDtypeStruct((NPAD, 1), jnp.float32)],
    )(dp_t, feat_pad, W1)

    p1 = _scatter128(x1s, src_r, dst_r, zeros128)

    x2s = pl.pallas_call(
        _l2_body,
        out_shape=jax.ShapeDtypeStruct((NPAD, 128), jnp.float32),
    )(p1, x1s, dinv, b1.reshape(1, 128), W2)

    p2 = _scatter128(x2s, src_r, dst_r, zeros128)

    b2p = jnp.concatenate([b2, jnp.zeros((64,), jnp.float32)]).reshape(1, 128)
    h2 = pl.pallas_call(
        _h2_body,
        out_shape=jax.ShapeDtypeStruct((NPAD, 128), jnp.float32),
    )(p2, x2s, dinv, b2p)

    pg = _pair_kernel(h2, i0_r, i1_r)

    dW1p = jnp.concatenate([dW1, jnp.zeros((64, 16), jnp.float32)], axis=0)
    BH = 12800
    o2d = pl.pallas_call(
        _dec_body,
        grid=(PPAD // BH,),
        in_specs=[
            pl.BlockSpec((BH, 128), lambda i: (i, 0)),
            pl.BlockSpec((BH, 128), lambda i: (i, 0)),
            pl.BlockSpec((128, 16), lambda i: (0, 0)),
            pl.BlockSpec((1, 16), lambda i: (0, 0)),
            pl.BlockSpec((16, 1), lambda i: (0, 0)),
            pl.BlockSpec((1, 1), lambda i: (0, 0)),
        ],
        out_specs=pl.BlockSpec((BH, 1), lambda i: (i, 0)),
        out_shape=jax.ShapeDtypeStruct((PPAD, 1), jnp.float32),
    )(pg[0], pg[1], dW1p, db1.reshape(1, 16), dW2, db2.reshape(1, 1))

    return o2d[:P, 0]


# speedup vs baseline: 11.3793x; 2.5469x over previous
import functools

import jax
import jax.numpy as jnp
from jax import lax
from jax.experimental import pallas as pl
from jax.experimental.pallas import tpu as pltpu
from jax.experimental.pallas import tpu_sc as plsc

N = 10000
NPAD = 10240
E = 320000
NW = 32
ECH = 80
EPT = ECH * 128
EPAD = NW * EPT
P = 100000
PCH = 25
PPT = PCH * 128
PPAD = NW * PPT
DEGW = 16
STRIPE = NPAD // 16

_mesh = lambda: plsc.VectorSubcoreMesh(core_axis_name="c", subcore_axis_name="s")


def _make_scatter_kernel(D):
    @functools.partial(
        pl.kernel,
        mesh=_mesh(),
        out_type=jax.ShapeDtypeStruct((2, NPAD, D), jnp.float32),
        scratch_types=[
            pltpu.VMEM((ECH, 128), jnp.int32),
            pltpu.VMEM((ECH, 128), jnp.int32),
            pltpu.VMEM((128, D), jnp.float32),
            pltpu.VMEM_SHARED((NPAD, D), jnp.float32),
            pltpu.SemaphoreType.DMA,
        ],
    )
    def k(table_hbm, src_hbm, dst_hbm, zeros_hbm, out_hbm,
          src_v, dst_v, rows_v, acc_s, sem):
        cid = lax.axis_index("c")
        sid = lax.axis_index("s")
        wid = cid * 16 + sid
        pltpu.sync_copy(src_hbm.at[wid], src_v)
        pltpu.sync_copy(dst_hbm.at[wid], dst_v)
        pltpu.sync_copy(zeros_hbm, rows_v)
        for kk in range(STRIPE // 128):
            pltpu.sync_copy(rows_v, acc_s.at[pl.ds(sid * STRIPE + kk * 128, 128)])
        plsc.subcore_barrier()

        def body(j, carry):
            pltpu.async_copy(table_hbm.at[src_v.at[j]], rows_v, sem).wait()
            pltpu.sync_copy(rows_v, acc_s.at[dst_v.at[j]], add=True)
            return carry

        lax.fori_loop(0, ECH, body, 0)
        plsc.subcore_barrier()
        for kk in range(STRIPE // 128):
            pltpu.sync_copy(acc_s.at[pl.ds(sid * STRIPE + kk * 128, 128)], rows_v)
            pltpu.sync_copy(rows_v, out_hbm.at[cid, pl.ds(sid * STRIPE + kk * 128, 128)])

    return k


_scatter128 = _make_scatter_kernel(128)


@functools.partial(
    pl.kernel,
    mesh=_mesh(),
    out_type=jax.ShapeDtypeStruct((2, PPAD, 128), jnp.float32),
    scratch_types=[
        pltpu.VMEM((PCH, 128), jnp.int32),
        pltpu.VMEM((PCH, 128), jnp.int32),
        pltpu.VMEM((128, 128), jnp.float32),
        pltpu.SemaphoreType.DMA,
    ],
)
def _pair_kernel(table_hbm, i0_hbm, i1_hbm, out_hbm, i0_v, i1_v, rows_v, sem):
    cid = lax.axis_index("c")
    sid = lax.axis_index("s")
    wid = cid * 16 + sid
    base = wid * PPT
    pltpu.sync_copy(i0_hbm.at[wid], i0_v)
    pltpu.sync_copy(i1_hbm.at[wid], i1_v)

    def body(j, carry):
        pltpu.async_copy(table_hbm.at[i0_v.at[j]], rows_v, sem).wait()
        pltpu.sync_copy(rows_v, out_hbm.at[0, pl.ds(base + j * 128, 128)])
        pltpu.async_copy(table_hbm.at[i1_v.at[j]], rows_v, sem).wait()
        pltpu.sync_copy(rows_v, out_hbm.at[1, pl.ds(base + j * 128, 128)])
        return carry

    lax.fori_loop(0, PCH, body, 0)


def _mm1_body(dp_ref, f_ref, w_ref, x1s_ref, dinv_ref):
    deg = dp_ref[:, 0:1] + dp_ref[:, 1:2] + 1.0
    dinv = lax.rsqrt(jnp.maximum(deg, 1e-12))
    x1 = jnp.dot(f_ref[...], w_ref[...], preferred_element_type=jnp.float32)
    x1s_ref[...] = x1 * dinv
    dinv_ref[...] = dinv


def _l2_body(p_ref, x1s_ref, dinv_ref, b1_ref, w2_ref, x2s_ref):
    s = p_ref[0] + p_ref[1] + x1s_ref[...]
    h1 = jnp.maximum(dinv_ref[...] * s + b1_ref[...], 0.0)
    y = jnp.dot(h1, w2_ref[...], preferred_element_type=jnp.float32) * dinv_ref[...]
    x2s_ref[...] = jnp.concatenate([y, jnp.zeros_like(y)], axis=1)


def _h2_body(q_ref, x2s_ref, dinv_ref, b2_ref, h2_ref):
    s = q_ref[0] + q_ref[1] + x2s_ref[...]
    h2_ref[...] = jnp.maximum(dinv_ref[...] * s + b2_ref[...], 0.0)


def _dec_body(xj_ref, xi_ref, dw1_ref, db1_ref, dw2_ref, db2_ref, o_ref):
    h = jnp.abs(xj_ref[...] - xi_ref[...])
    g = jax.nn.sigmoid(
        jnp.dot(h, dw1_ref[...], preferred_element_type=jnp.float32) + db1_ref[...])
    o_ref[...] = jnp.dot(g, dw2_ref[...], preferred_element_type=jnp.float32) + db2_ref[...]


def _pad_reshape_idx(a, total, per_tile_chunks, base, mod):
    pad = total - a.shape[0]
    fill = base + jnp.arange(pad, dtype=jnp.int32) % mod
    a32 = a.astype(jnp.int32)
    return jnp.concatenate([a32, fill]).reshape(NW, per_tile_chunks, 128)


def kernel(features, adj, idx, W1, b1, W2, b2, dW1, db1, dW2, db2):
    src_r = _pad_reshape_idx(adj[0], EPAD, ECH, 0, N)
    dst_r = _pad_reshape_idx(adj[1], EPAD, ECH, N, NPAD - N)
    i0_r = _pad_reshape_idx(idx[0], PPAD, PCH, 0, N)
    i1_r = _pad_reshape_idx(idx[1], PPAD, PCH, 0, N)
    feat_pad = jnp.pad(features, ((0, NPAD - N), (0, 0)))

    zeros128 = jnp.zeros((128, 128), jnp.float32)
    ones_table = jnp.ones((NPAD, 128), jnp.float32)

    deg_parts = _scatter128(ones_table, src_r, dst_r, zeros128)
    dp_t = jnp.transpose(deg_parts[:, :, 0])

    x1s, dinv = pl.pallas_call(
        _mm1_body,
        out_shape=[jax.ShapeDtypeStruct((NPAD, 128), jnp.float32),
                   jax.ShapeDtypeStruct((NPAD, 1), jnp.float32)],
    )(dp_t, feat_pad, W1)

    p1 = _scatter128(x1s, src_r, dst_r, zeros128)

    x2s = pl.pallas_call(
        _l2_body,
        out_shape=jax.ShapeDtypeStruct((NPAD, 128), jnp.float32),
    )(p1, x1s, dinv, b1.reshape(1, 128), W2)

    p2 = _scatter128(x2s, src_r, dst_r, zeros128)

    b2p = jnp.concatenate([b2, jnp.zeros((64,), jnp.float32)]).reshape(1, 128)
    h2 = pl.pallas_call(
        _h2_body,
        out_shape=jax.ShapeDtypeStruct((NPAD, 128), jnp.float32),
    )(p2, x2s, dinv, b2p)

    pg = _pair_kernel(h2, i0_r, i1_r)

    dW1p = jnp.concatenate([dW1, jnp.zeros((64, 16), jnp.float32)], axis=0)
    BH = 12800
    o2d = pl.pallas_call(
        _dec_body,
        grid=(PPAD // BH,),
        in_specs=[
            pl.BlockSpec((BH, 128), lambda i: (i, 0)),
            pl.BlockSpec((BH, 128), lambda i: (i, 0)),
            pl.BlockSpec((128, 16), lambda i: (0, 0)),
            pl.BlockSpec((1, 16), lambda i: (0, 0)),
            pl.BlockSpec((16, 1), lambda i: (0, 0)),
            pl.BlockSpec((1, 1), lambda i: (0, 0)),
        ],
        out_specs=pl.BlockSpec((BH, 1), lambda i: (i, 0)),
        out_shape=jax.ShapeDtypeStruct((PPAD, 1), jnp.float32),
    )(pg[0], pg[1], dW1p, db1.reshape(1, 16), dW2, db2.reshape(1, 1))

    return o2d[:P, 0]


# speedup vs baseline: 15.2315x; 1.3385x over previous
import functools

import jax
import jax.numpy as jnp
from jax import lax
from jax.experimental import pallas as pl
from jax.experimental.pallas import tpu as pltpu
from jax.experimental.pallas import tpu_sc as plsc

N = 10000
NPAD = 10240
E = 320000
NW = 32
ECH = 80
EPT = ECH * 128
EPAD = NW * EPT
P = 100000
PCH = 25
PPT = PCH * 128
PPAD = NW * PPT
DEGW = 16
STRIPE = NPAD // 16

_mesh = lambda: plsc.VectorSubcoreMesh(core_axis_name="c", subcore_axis_name="s")


@functools.partial(
    pl.kernel,
    mesh=_mesh(),
    out_type=jax.ShapeDtypeStruct((NW, NPAD), jnp.float32),
    scratch_types=[
        pltpu.VMEM((EPT,), jnp.int32),
        pltpu.VMEM((NPAD,), jnp.float32),
    ],
    compiler_params=pltpu.CompilerParams(needs_layout_passes=False),
)
def _deg_hist(dst_hbm, zeros_hbm, out_hbm, idx_v, hist_v):
    cid = lax.axis_index("c")
    sid = lax.axis_index("s")
    wid = cid * 16 + sid
    pltpu.sync_copy(dst_hbm.at[wid], idx_v)
    pltpu.sync_copy(zeros_hbm, hist_v)
    ones = jnp.ones((16,), jnp.float32)

    @pl.loop(0, EPT // 16)
    def _hist(i):
        plsc.addupdate_scatter(hist_v, [idx_v[pl.ds(i * 16, 16)]], ones)

    pltpu.sync_copy(hist_v, out_hbm.at[wid])


def _make_scatter_kernel(D):
    @functools.partial(
        pl.kernel,
        mesh=_mesh(),
        out_type=jax.ShapeDtypeStruct((2, NPAD, D), jnp.float32),
        scratch_types=[
            pltpu.VMEM((ECH, 128), jnp.int32),
            pltpu.VMEM((ECH, 128), jnp.int32),
            pltpu.VMEM((128, D), jnp.float32),
            pltpu.VMEM_SHARED((NPAD, D), jnp.float32),
            pltpu.SemaphoreType.DMA,
        ],
    )
    def k(table_hbm, src_hbm, dst_hbm, zeros_hbm, out_hbm,
          src_v, dst_v, rows_v, acc_s, sem):
        cid = lax.axis_index("c")
        sid = lax.axis_index("s")
        wid = cid * 16 + sid
        pltpu.sync_copy(src_hbm.at[wid], src_v)
        pltpu.sync_copy(dst_hbm.at[wid], dst_v)
        pltpu.sync_copy(zeros_hbm, rows_v)
        for kk in range(STRIPE // 128):
            pltpu.sync_copy(rows_v, acc_s.at[pl.ds(sid * STRIPE + kk * 128, 128)])
        plsc.subcore_barrier()

        def body(j, carry):
            pltpu.async_copy(table_hbm.at[src_v.at[j]], rows_v, sem).wait()
            pltpu.sync_copy(rows_v, acc_s.at[dst_v.at[j]], add=True)
            return carry

        lax.fori_loop(0, ECH, body, 0)
        plsc.subcore_barrier()
        for kk in range(STRIPE // 128):
            pltpu.sync_copy(acc_s.at[pl.ds(sid * STRIPE + kk * 128, 128)], rows_v)
            pltpu.sync_copy(rows_v, out_hbm.at[cid, pl.ds(sid * STRIPE + kk * 128, 128)])

    return k


_scatter128 = _make_scatter_kernel(128)


@functools.partial(
    pl.kernel,
    mesh=_mesh(),
    out_type=jax.ShapeDtypeStruct((2, PPAD, 128), jnp.float32),
    scratch_types=[
        pltpu.VMEM((PCH, 128), jnp.int32),
        pltpu.VMEM((PCH, 128), jnp.int32),
        pltpu.VMEM((128, 128), jnp.float32),
        pltpu.SemaphoreType.DMA,
    ],
)
def _pair_kernel(table_hbm, i0_hbm, i1_hbm, out_hbm, i0_v, i1_v, rows_v, sem):
    cid = lax.axis_index("c")
    sid = lax.axis_index("s")
    wid = cid * 16 + sid
    base = wid * PPT
    pltpu.sync_copy(i0_hbm.at[wid], i0_v)
    pltpu.sync_copy(i1_hbm.at[wid], i1_v)

    def body(j, carry):
        pltpu.async_copy(table_hbm.at[i0_v.at[j]], rows_v, sem).wait()
        pltpu.sync_copy(rows_v, out_hbm.at[0, pl.ds(base + j * 128, 128)])
        pltpu.async_copy(table_hbm.at[i1_v.at[j]], rows_v, sem).wait()
        pltpu.sync_copy(rows_v, out_hbm.at[1, pl.ds(base + j * 128, 128)])
        return carry

    lax.fori_loop(0, PCH, body, 0)


def _mm1_body(dp_ref, f_ref, w_ref, x1s_ref, dinv_ref):
    deg = lax.dot_general(dp_ref[...], jnp.ones((NW, 1), jnp.float32),
                          (((0,), (0,)), ((), ())),
                          preferred_element_type=jnp.float32) + 1.0
    dinv = lax.rsqrt(jnp.maximum(deg, 1e-12))
    x1 = jnp.dot(f_ref[...], w_ref[...], preferred_element_type=jnp.float32)
    x1s_ref[...] = x1 * dinv
    dinv_ref[...] = dinv


def _l2_body(p_ref, x1s_ref, dinv_ref, b1_ref, w2_ref, x2s_ref):
    s = p_ref[0] + p_ref[1] + x1s_ref[...]
    h1 = jnp.maximum(dinv_ref[...] * s + b1_ref[...], 0.0)
    y = jnp.dot(h1, w2_ref[...], preferred_element_type=jnp.float32) * dinv_ref[...]
    x2s_ref[...] = jnp.concatenate([y, jnp.zeros_like(y)], axis=1)


def _h2_body(q_ref, x2s_ref, dinv_ref, b2_ref, h2_ref):
    s = q_ref[0] + q_ref[1] + x2s_ref[...]
    h2_ref[...] = jnp.maximum(dinv_ref[...] * s + b2_ref[...], 0.0)


def _dec_body(xj_ref, xi_ref, dw1_ref, db1_ref, dw2_ref, db2_ref, o_ref):
    h = jnp.abs(xj_ref[...] - xi_ref[...])
    g = jax.nn.sigmoid(
        jnp.dot(h, dw1_ref[...], preferred_element_type=jnp.float32) + db1_ref[...])
    o_ref[...] = jnp.dot(g, dw2_ref[...], preferred_element_type=jnp.float32) + db2_ref[...]


def _pad_reshape_idx(a, total, per_tile_chunks, base, mod):
    pad = total - a.shape[0]
    fill = base + jnp.arange(pad, dtype=jnp.int32) % mod
    a32 = a.astype(jnp.int32)
    return jnp.concatenate([a32, fill]).reshape(NW, per_tile_chunks, 128)


def kernel(features, adj, idx, W1, b1, W2, b2, dW1, db1, dW2, db2):
    src_r = _pad_reshape_idx(adj[0], EPAD, ECH, 0, N)
    dst_r = _pad_reshape_idx(adj[1], EPAD, ECH, N, NPAD - N)
    i0_r = _pad_reshape_idx(idx[0], PPAD, PCH, 0, N)
    i1_r = _pad_reshape_idx(idx[1], PPAD, PCH, 0, N)
    feat_pad = jnp.pad(features, ((0, NPAD - N), (0, 0)))

    zeros128 = jnp.zeros((128, 128), jnp.float32)
    zeros_n = jnp.zeros((NPAD,), jnp.float32)

    dst_h = dst_r.reshape(NW, EPT)
    dp = _deg_hist(dst_h, zeros_n)

    x1s, dinv = pl.pallas_call(
        _mm1_body,
        out_shape=[jax.ShapeDtypeStruct((NPAD, 128), jnp.float32),
                   jax.ShapeDtypeStruct((NPAD, 1), jnp.float32)],
    )(dp, feat_pad, W1)

    p1 = _scatter128(x1s, src_r, dst_r, zeros128)

    x2s = pl.pallas_call(
        _l2_body,
        out_shape=jax.ShapeDtypeStruct((NPAD, 128), jnp.float32),
    )(p1, x1s, dinv, b1.reshape(1, 128), W2)

    p2 = _scatter128(x2s, src_r, dst_r, zeros128)

    b2p = jnp.concatenate([b2, jnp.zeros((64,), jnp.float32)]).reshape(1, 128)
    h2 = pl.pallas_call(
        _h2_body,
        out_shape=jax.ShapeDtypeStruct((NPAD, 128), jnp.float32),
    )(p2, x2s, dinv, b2p)

    pg = _pair_kernel(h2, i0_r, i1_r)

    dW1p = jnp.concatenate([dW1, jnp.zeros((64, 16), jnp.float32)], axis=0)
    BH = 12800
    o2d = pl.pallas_call(
        _dec_body,
        grid=(PPAD // BH,),
        in_specs=[
            pl.BlockSpec((BH, 128), lambda i: (i, 0)),
            pl.BlockSpec((BH, 128), lambda i: (i, 0)),
            pl.BlockSpec((128, 16), lambda i: (0, 0)),
            pl.BlockSpec((1, 16), lambda i: (0, 0)),
            pl.BlockSpec((16, 1), lambda i: (0, 0)),
            pl.BlockSpec((1, 1), lambda i: (0, 0)),
        ],
        out_specs=pl.BlockSpec((BH, 1), lambda i: (i, 0)),
        out_shape=jax.ShapeDtypeStruct((PPAD, 1), jnp.float32),
    )(pg[0], pg[1], dW1p, db1.reshape(1, 16), dW2, db2.reshape(1, 1))

    return o2d[:P, 0]


# speedup vs baseline: 19.4733x; 1.2785x over previous
import functools

import jax
import jax.numpy as jnp
from jax import lax
from jax.experimental import pallas as pl
from jax.experimental.pallas import tpu as pltpu
from jax.experimental.pallas import tpu_sc as plsc

N = 10000
NPAD = 10240
E = 320000
NW = 32
ECH = 80
EPT = ECH * 128
EPAD = NW * EPT
P = 100000
PCH = 25
PPT = PCH * 128
PPAD = NW * PPT
DEGW = 16
STRIPE = NPAD // 16

_mesh = lambda: plsc.VectorSubcoreMesh(core_axis_name="c", subcore_axis_name="s")


@functools.partial(
    pl.kernel,
    mesh=_mesh(),
    out_type=jax.ShapeDtypeStruct((NW, NPAD), jnp.float32),
    scratch_types=[
        pltpu.VMEM((EPT,), jnp.int32),
        pltpu.VMEM((NPAD,), jnp.float32),
    ],
    compiler_params=pltpu.CompilerParams(needs_layout_passes=False),
)
def _deg_hist(dst_hbm, zeros_hbm, out_hbm, idx_v, hist_v):
    cid = lax.axis_index("c")
    sid = lax.axis_index("s")
    wid = cid * 16 + sid
    pltpu.sync_copy(dst_hbm.at[wid], idx_v)
    pltpu.sync_copy(zeros_hbm, hist_v)
    ones = jnp.ones((16,), jnp.float32)

    @pl.loop(0, EPT // 16)
    def _hist(i):
        plsc.addupdate_scatter(hist_v, [idx_v[pl.ds(i * 16, 16)]], ones)

    pltpu.sync_copy(hist_v, out_hbm.at[wid])


def _make_scatter_kernel(D):
    PASSES = 5
    PCHK = ECH // PASSES

    @functools.partial(
        pl.kernel,
        mesh=_mesh(),
        out_type=jax.ShapeDtypeStruct((2, NPAD, D), jnp.float32),
        scratch_types=[
            pltpu.VMEM((PCHK, 128), jnp.int32),
            pltpu.VMEM((PCHK, 128), jnp.int32),
            pltpu.VMEM((128, D), jnp.float32),
            pltpu.VMEM((128, D), jnp.float32),
            pltpu.VMEM_SHARED((NPAD, D), jnp.float32),
            pltpu.SemaphoreType.DMA,
            pltpu.SemaphoreType.DMA,
        ],
    )
    def k(table_hbm, src_hbm, dst_hbm, zeros_hbm, out_hbm,
          src_v, dst_v, rows_a, rows_b, acc_s, sem_a, sem_b):
        cid = lax.axis_index("c")
        sid = lax.axis_index("s")
        wid = cid * 16 + sid
        pltpu.sync_copy(zeros_hbm, rows_a)
        for kk in range(STRIPE // 128):
            pltpu.sync_copy(rows_a, acc_s.at[pl.ds(sid * STRIPE + kk * 128, 128)])
        plsc.subcore_barrier()

        for p in range(PASSES):
            pltpu.sync_copy(src_hbm.at[wid, pl.ds(p * PCHK, PCHK)], src_v)
            pltpu.sync_copy(dst_hbm.at[wid, pl.ds(p * PCHK, PCHK)], dst_v)
            pltpu.async_copy(table_hbm.at[src_v.at[0]], rows_a, sem_a)

            @pl.loop(0, PCHK, step=2)
            def _mp(j):
                pltpu.async_copy(table_hbm.at[src_v.at[j + 1]], rows_b, sem_b)
                pltpu.make_async_copy(table_hbm.at[src_v.at[j]], rows_a, sem_a).wait()
                pltpu.sync_copy(rows_a, acc_s.at[dst_v.at[j]], add=True)

                @pl.when(j + 2 < PCHK)
                def _fire():
                    pltpu.async_copy(table_hbm.at[src_v.at[j + 2]], rows_a, sem_a)

                pltpu.make_async_copy(table_hbm.at[src_v.at[j + 1]], rows_b, sem_b).wait()
                pltpu.sync_copy(rows_b, acc_s.at[dst_v.at[j + 1]], add=True)

        plsc.subcore_barrier()
        for kk in range(STRIPE // 128):
            pltpu.sync_copy(acc_s.at[pl.ds(sid * STRIPE + kk * 128, 128)], rows_a)
            pltpu.sync_copy(rows_a, out_hbm.at[cid, pl.ds(sid * STRIPE + kk * 128, 128)])

    return k


_scatter128 = _make_scatter_kernel(128)


@functools.partial(
    pl.kernel,
    mesh=_mesh(),
    out_type=jax.ShapeDtypeStruct((2, PPAD, 128), jnp.float32),
    scratch_types=[
        pltpu.VMEM((PCH, 128), jnp.int32),
        pltpu.VMEM((PCH, 128), jnp.int32),
        pltpu.VMEM((128, 128), jnp.float32),
        pltpu.VMEM((128, 128), jnp.float32),
        pltpu.SemaphoreType.DMA,
        pltpu.SemaphoreType.DMA,
    ],
)
def _pair_kernel(table_hbm, i0_hbm, i1_hbm, out_hbm, i0_v, i1_v,
                 rows_a, rows_b, sem_a, sem_b):
    cid = lax.axis_index("c")
    sid = lax.axis_index("s")
    wid = cid * 16 + sid
    base = wid * PPT
    pltpu.sync_copy(i0_hbm.at[wid], i0_v)
    pltpu.sync_copy(i1_hbm.at[wid], i1_v)
    pltpu.async_copy(table_hbm.at[i0_v.at[0]], rows_a, sem_a)

    @pl.loop(0, PCH)
    def _pg(j):
        pltpu.async_copy(table_hbm.at[i1_v.at[j]], rows_b, sem_b)
        pltpu.make_async_copy(table_hbm.at[i0_v.at[j]], rows_a, sem_a).wait()
        pltpu.sync_copy(rows_a, out_hbm.at[0, pl.ds(base + j * 128, 128)])
        @pl.when(j + 1 < PCH)
        def _fire():
            pltpu.async_copy(table_hbm.at[i0_v.at[j + 1]], rows_a, sem_a)

        pltpu.make_async_copy(table_hbm.at[i1_v.at[j]], rows_b, sem_b).wait()
        pltpu.sync_copy(rows_b, out_hbm.at[1, pl.ds(base + j * 128, 128)])


def _mm1_body(dp_ref, f_ref, w_ref, x1s_ref, dinv_ref):
    deg = lax.dot_general(dp_ref[...], jnp.ones((NW, 1), jnp.float32),
                          (((0,), (0,)), ((), ())),
                          preferred_element_type=jnp.float32) + 1.0
    dinv = lax.rsqrt(jnp.maximum(deg, 1e-12))
    x1 = jnp.dot(f_ref[...], w_ref[...], preferred_element_type=jnp.float32)
    x1s_ref[...] = x1 * dinv
    dinv_ref[...] = dinv


def _l2_body(p_ref, x1s_ref, dinv_ref, b1_ref, w2_ref, x2s_ref):
    s = p_ref[0] + p_ref[1] + x1s_ref[...]
    h1 = jnp.maximum(dinv_ref[...] * s + b1_ref[...], 0.0)
    y = jnp.dot(h1, w2_ref[...], preferred_element_type=jnp.float32) * dinv_ref[...]
    x2s_ref[...] = jnp.concatenate([y, jnp.zeros_like(y)], axis=1)


def _h2_body(q_ref, x2s_ref, dinv_ref, b2_ref, h2_ref):
    s = q_ref[0] + q_ref[1] + x2s_ref[...]
    h2_ref[...] = jnp.maximum(dinv_ref[...] * s + b2_ref[...], 0.0)


def _dec_body(xj_ref, xi_ref, dw1_ref, db1_ref, dw2_ref, db2_ref, o_ref):
    h = jnp.abs(xj_ref[...] - xi_ref[...])
    g = jax.nn.sigmoid(
        jnp.dot(h, dw1_ref[...], preferred_element_type=jnp.float32) + db1_ref[...])
    o_ref[...] = jnp.dot(g, dw2_ref[...], preferred_element_type=jnp.float32) + db2_ref[...]


def _pad_reshape_idx(a, total, per_tile_chunks, base, mod):
    pad = total - a.shape[0]
    fill = base + jnp.arange(pad, dtype=jnp.int32) % mod
    a32 = a.astype(jnp.int32)
    return jnp.concatenate([a32, fill]).reshape(NW, per_tile_chunks, 128)


def kernel(features, adj, idx, W1, b1, W2, b2, dW1, db1, dW2, db2):
    src_r = _pad_reshape_idx(adj[0], EPAD, ECH, 0, N)
    dst_r = _pad_reshape_idx(adj[1], EPAD, ECH, N, NPAD - N)
    i0_r = _pad_reshape_idx(idx[0], PPAD, PCH, 0, N)
    i1_r = _pad_reshape_idx(idx[1], PPAD, PCH, 0, N)
    feat_pad = jnp.pad(features, ((0, NPAD - N), (0, 0)))

    zeros128 = jnp.zeros((128, 128), jnp.float32)
    zeros_n = jnp.zeros((NPAD,), jnp.float32)

    dst_h = dst_r.reshape(NW, EPT)
    dp = _deg_hist(dst_h, zeros_n)

    x1s, dinv = pl.pallas_call(
        _mm1_body,
        out_shape=[jax.ShapeDtypeStruct((NPAD, 128), jnp.float32),
                   jax.ShapeDtypeStruct((NPAD, 1), jnp.float32)],
    )(dp, feat_pad, W1)

    p1 = _scatter128(x1s, src_r, dst_r, zeros128)

    x2s = pl.pallas_call(
        _l2_body,
        out_shape=jax.ShapeDtypeStruct((NPAD, 128), jnp.float32),
    )(p1, x1s, dinv, b1.reshape(1, 128), W2)

    p2 = _scatter128(x2s, src_r, dst_r, zeros128)

    b2p = jnp.concatenate([b2, jnp.zeros((64,), jnp.float32)]).reshape(1, 128)
    h2 = pl.pallas_call(
        _h2_body,
        out_shape=jax.ShapeDtypeStruct((NPAD, 128), jnp.float32),
    )(p2, x2s, dinv, b2p)

    pg = _pair_kernel(h2, i0_r, i1_r)

    dW1p = jnp.concatenate([dW1, jnp.zeros((64, 16), jnp.float32)], axis=0)
    BH = 12800
    o2d = pl.pallas_call(
        _dec_body,
        grid=(PPAD // BH,),
        in_specs=[
            pl.BlockSpec((BH, 128), lambda i: (i, 0)),
            pl.BlockSpec((BH, 128), lambda i: (i, 0)),
            pl.BlockSpec((128, 16), lambda i: (0, 0)),
            pl.BlockSpec((1, 16), lambda i: (0, 0)),
            pl.BlockSpec((16, 1), lambda i: (0, 0)),
            pl.BlockSpec((1, 1), lambda i: (0, 0)),
        ],
        out_specs=pl.BlockSpec((BH, 1), lambda i: (i, 0)),
        out_shape=jax.ShapeDtypeStruct((PPAD, 1), jnp.float32),
    )(pg[0], pg[1], dW1p, db1.reshape(1, 16), dW2, db2.reshape(1, 1))

    return o2d[:P, 0]


# speedup vs baseline: 19.4760x; 1.0001x over previous
import functools

import jax
import jax.numpy as jnp
from jax import lax
from jax.experimental import pallas as pl
from jax.experimental.pallas import tpu as pltpu
from jax.experimental.pallas import tpu_sc as plsc

N = 10000
NPAD = 10240
E = 320000
NW = 32
ECH = 80
EPT = ECH * 128
EPAD = NW * EPT
P = 100000
PCH = 25
PPT = PCH * 128
PPAD = NW * PPT
STRIPE = NPAD // 16

_mesh = lambda: plsc.VectorSubcoreMesh(core_axis_name="c", subcore_axis_name="s")


@functools.partial(
    pl.kernel,
    mesh=_mesh(),
    out_type=jax.ShapeDtypeStruct((NW, NPAD), jnp.float32),
    scratch_types=[
        pltpu.VMEM((EPT,), jnp.int32),
        pltpu.VMEM((NPAD,), jnp.float32),
    ],
    compiler_params=pltpu.CompilerParams(needs_layout_passes=False),
)
def _deg_hist(dst_hbm, zeros_hbm, out_hbm, idx_v, hist_v):
    cid = lax.axis_index("c")
    sid = lax.axis_index("s")
    wid = cid * 16 + sid
    pltpu.sync_copy(dst_hbm.at[wid], idx_v)
    pltpu.sync_copy(zeros_hbm, hist_v)
    ones = jnp.ones((16,), jnp.float32)

    @pl.loop(0, EPT // 16)
    def _hist(i):
        plsc.addupdate_scatter(hist_v, [idx_v[pl.ds(i * 16, 16)]], ones)

    pltpu.sync_copy(hist_v, out_hbm.at[wid])


def _make_scatter_kernel(D):
    PASSES = 5
    PCHK = ECH // PASSES

    @functools.partial(
        pl.kernel,
        mesh=_mesh(),
        out_type=jax.ShapeDtypeStruct((2, NPAD, D), jnp.float32),
        scratch_types=[
            pltpu.VMEM((PCHK, 128), jnp.int32),
            pltpu.VMEM((PCHK, 128), jnp.int32),
            pltpu.VMEM((128, D), jnp.float32),
            pltpu.VMEM((128, D), jnp.float32),
            pltpu.VMEM_SHARED((NPAD, D), jnp.float32),
            pltpu.SemaphoreType.DMA,
            pltpu.SemaphoreType.DMA,
        ],
    )
    def k(table_hbm, src_hbm, dst_hbm, zeros_hbm, out_hbm,
          src_v, dst_v, rows_a, rows_b, acc_s, sem_a, sem_b):
        cid = lax.axis_index("c")
        sid = lax.axis_index("s")
        wid = cid * 16 + sid
        pltpu.sync_copy(zeros_hbm, rows_a)
        for kk in range(STRIPE // 128):
            pltpu.sync_copy(rows_a, acc_s.at[pl.ds(sid * STRIPE + kk * 128, 128)])
        plsc.subcore_barrier()

        for p in range(PASSES):
            pltpu.sync_copy(src_hbm.at[wid, pl.ds(p * PCHK, PCHK)], src_v)
            pltpu.sync_copy(dst_hbm.at[wid, pl.ds(p * PCHK, PCHK)], dst_v)
            pltpu.async_copy(table_hbm.at[src_v.at[0]], rows_a, sem_a)

            @pl.loop(0, PCHK, step=2)
            def _mp(j):
                pltpu.async_copy(table_hbm.at[src_v.at[j + 1]], rows_b, sem_b)
                pltpu.make_async_copy(table_hbm.at[src_v.at[j]], rows_a, sem_a).wait()
                pltpu.sync_copy(rows_a, acc_s.at[dst_v.at[j]], add=True)

                @pl.when(j + 2 < PCHK)
                def _fire():
                    pltpu.async_copy(table_hbm.at[src_v.at[j + 2]], rows_a, sem_a)

                pltpu.make_async_copy(table_hbm.at[src_v.at[j + 1]], rows_b, sem_b).wait()
                pltpu.sync_copy(rows_b, acc_s.at[dst_v.at[j + 1]], add=True)

        plsc.subcore_barrier()
        for kk in range(STRIPE // 128):
            pltpu.sync_copy(acc_s.at[pl.ds(sid * STRIPE + kk * 128, 128)], rows_a)
            pltpu.sync_copy(rows_a, out_hbm.at[cid, pl.ds(sid * STRIPE + kk * 128, 128)])

    return k


_scatter128 = _make_scatter_kernel(128)


@functools.partial(
    pl.kernel,
    mesh=_mesh(),
    out_type=jax.ShapeDtypeStruct((2, PPAD, 128), jnp.float32),
    scratch_types=[
        pltpu.VMEM((PCH, 128), jnp.int32),
        pltpu.VMEM((PCH, 128), jnp.int32),
        pltpu.VMEM((128, 128), jnp.float32),
        pltpu.VMEM((128, 128), jnp.float32),
        pltpu.SemaphoreType.DMA,
        pltpu.SemaphoreType.DMA,
    ],
)
def _pair_kernel(table_hbm, i0_hbm, i1_hbm, out_hbm, i0_v, i1_v,
                 rows_a, rows_b, sem_a, sem_b):
    cid = lax.axis_index("c")
    sid = lax.axis_index("s")
    wid = cid * 16 + sid
    base = wid * PPT
    pltpu.sync_copy(i0_hbm.at[wid], i0_v)
    pltpu.sync_copy(i1_hbm.at[wid], i1_v)
    pltpu.async_copy(table_hbm.at[i0_v.at[0]], rows_a, sem_a)

    @pl.loop(0, PCH)
    def _pg(j):
        pltpu.async_copy(table_hbm.at[i1_v.at[j]], rows_b, sem_b)
        pltpu.make_async_copy(table_hbm.at[i0_v.at[j]], rows_a, sem_a).wait()
        pltpu.sync_copy(rows_a, out_hbm.at[0, pl.ds(base + j * 128, 128)])
        @pl.when(j + 1 < PCH)
        def _fire():
            pltpu.async_copy(table_hbm.at[i0_v.at[j + 1]], rows_a, sem_a)

        pltpu.make_async_copy(table_hbm.at[i1_v.at[j]], rows_b, sem_b).wait()
        pltpu.sync_copy(rows_b, out_hbm.at[1, pl.ds(base + j * 128, 128)])


def _mm1_body(dp_ref, f_ref, w_ref, x1s_ref, dinv_ref):
    deg = lax.dot_general(dp_ref[...], jnp.ones((NW, 1), jnp.float32),
                          (((0,), (0,)), ((), ())),
                          preferred_element_type=jnp.float32) + 1.0
    dinv = lax.rsqrt(jnp.maximum(deg, 1e-12))
    x1 = jnp.dot(f_ref[...], w_ref[...], preferred_element_type=jnp.float32)
    x1s_ref[...] = x1 * dinv
    dinv_ref[...] = dinv


def _l2_body(p_ref, x1s_ref, dinv_ref, b1_ref, w2_ref, x2s_ref):
    s = p_ref[0] + p_ref[1] + x1s_ref[...]
    h1 = jnp.maximum(dinv_ref[...] * s + b1_ref[...], 0.0)
    y = jnp.dot(h1, w2_ref[...], preferred_element_type=jnp.float32) * dinv_ref[...]
    x2s_ref[...] = jnp.concatenate([y, jnp.zeros_like(y)], axis=1)


def _h2_body(q_ref, x2s_ref, dinv_ref, b2_ref, h2_ref):
    s = q_ref[0] + q_ref[1] + x2s_ref[...]
    h2_ref[...] = jnp.maximum(dinv_ref[...] * s + b2_ref[...], 0.0)


def _dec_body(xj_ref, xi_ref, dw1_ref, db1_ref, dw2_ref, db2_ref, o_ref):
    h = jnp.abs(xj_ref[...] - xi_ref[...])
    g = jax.nn.sigmoid(
        jnp.dot(h, dw1_ref[...], preferred_element_type=jnp.float32) + db1_ref[...])
    o_ref[...] = jnp.dot(g, dw2_ref[...], preferred_element_type=jnp.float32) + db2_ref[...]


def _pad_reshape_idx(a, total, per_tile_chunks, base, mod):
    pad = total - a.shape[0]
    fill = base + jnp.arange(pad, dtype=jnp.int32) % mod
    a32 = a.astype(jnp.int32)
    return jnp.concatenate([a32, fill]).reshape(NW, per_tile_chunks, 128)


def kernel(features, adj, idx, W1, b1, W2, b2, dW1, db1, dW2, db2):
    src_r = _pad_reshape_idx(adj[0], EPAD, ECH, 0, N)
    dst_r = _pad_reshape_idx(adj[1], EPAD, ECH, N, NPAD - N)
    i0_r = _pad_reshape_idx(idx[0], PPAD, PCH, 0, N)
    i1_r = _pad_reshape_idx(idx[1], PPAD, PCH, 0, N)
    feat_pad = jnp.pad(features, ((0, NPAD - N), (0, 0)))

    zeros128 = jnp.zeros((128, 128), jnp.float32)
    zeros_n = jnp.zeros((NPAD,), jnp.float32)

    dst_h = dst_r.reshape(NW, EPT)
    dp = _deg_hist(dst_h, zeros_n)

    x1s, dinv = pl.pallas_call(
        _mm1_body,
        out_shape=[jax.ShapeDtypeStruct((NPAD, 128), jnp.float32),
                   jax.ShapeDtypeStruct((NPAD, 1), jnp.float32)],
    )(dp, feat_pad, W1)

    p1 = _scatter128(x1s, src_r, dst_r, zeros128)

    x2s = pl.pallas_call(
        _l2_body,
        out_shape=jax.ShapeDtypeStruct((NPAD, 128), jnp.float32),
    )(p1, x1s, dinv, b1.reshape(1, 128), W2)

    p2 = _scatter128(x2s, src_r, dst_r, zeros128)

    b2p = jnp.concatenate([b2, jnp.zeros((64,), jnp.float32)]).reshape(1, 128)
    h2 = pl.pallas_call(
        _h2_body,
        out_shape=jax.ShapeDtypeStruct((NPAD, 128), jnp.float32),
    )(p2, x2s, dinv, b2p)

    pg = _pair_kernel(h2, i0_r, i1_r)

    dW1p = jnp.concatenate([dW1, jnp.zeros((64, 16), jnp.float32)], axis=0)
    BH = 12800
    o2d = pl.pallas_call(
        _dec_body,
        grid=(PPAD // BH,),
        in_specs=[
            pl.BlockSpec((BH, 128), lambda i: (i, 0)),
            pl.BlockSpec((BH, 128), lambda i: (i, 0)),
            pl.BlockSpec((128, 16), lambda i: (0, 0)),
            pl.BlockSpec((1, 16), lambda i: (0, 0)),
            pl.BlockSpec((16, 1), lambda i: (0, 0)),
            pl.BlockSpec((1, 1), lambda i: (0, 0)),
        ],
        out_specs=pl.BlockSpec((BH, 1), lambda i: (i, 0)),
        out_shape=jax.ShapeDtypeStruct((PPAD, 1), jnp.float32),
    )(pg[0], pg[1], dW1p, db1.reshape(1, 16), dW2, db2.reshape(1, 1))

    return o2d[:P, 0]
